# baseline (speedup 1.0000x reference)
import functools

import jax
import jax.numpy as jnp
from jax import lax
from jax.experimental import pallas as pl
from jax.experimental.pallas import tpu as pltpu
from jax.experimental.pallas import tpu_sc as plsc

N = 10000
NPAD = 10240
F = 128
NNZ = 320000
NC = 2
NS = 16
G = 8
CW = 125
IB = 16
PCH = NNZ // NC // CW
PNB = PCH // IB
HB = PCH // NS // IB
GW = NPAD * G
HROW = NPAD // NC
AR = N // NC

_MESH = plsc.VectorSubcoreMesh(core_axis_name="c", subcore_axis_name="s")
_SC_PARAMS = pltpu.CompilerParams(use_tc_tiling_on_sc=False)


def _matmul_body(x_ref, w_ref, o_ref):
    o_ref[...] = lax.dot_general(
        x_ref[...], w_ref[...], (((1,), (1,)), ((), ())),
        preferred_element_type=jnp.float32)


def _matmul(x, w):
    return pl.pallas_call(
        _matmul_body,
        out_shape=jax.ShapeDtypeStruct((NPAD, F), jnp.float32),
    )(x, w)


def _split_body(xt2, tab3, v2d):
    c = lax.axis_index("c")
    s = lax.axis_index("s")
    r0 = c * HROW
    pltpu.sync_copy(xt2.at[pl.ds(r0, HROW), pl.ds(G * s, G)], v2d)
    pltpu.sync_copy(v2d, tab3.at[s, pl.ds(r0, HROW)])


_split = functools.partial(
    pl.kernel,
    out_type=jax.ShapeDtypeStruct((NS, NPAD, G), jnp.float32),
    mesh=_MESH,
    compiler_params=_SC_PARAMS,
    scratch_types=[
        pltpu.VMEM((HROW, G), jnp.float32),
    ],
)(_split_body)


def _hist_body(row_idx, col_idx, zeros_g, ones_g, part_out, iv, obuf, acc):
    c = lax.axis_index("c")
    s = lax.axis_index("s")
    pltpu.sync_copy(zeros_g, acc.at[s])
    pltpu.sync_copy(ones_g, obuf)
    acc_g = acc.at[s]

    def mk_block(idx, h):
        def block(b, carry):
            pltpu.sync_copy(idx.at[h, pl.ds(s * (PCH // NS) + b * IB, IB)],
                            iv)

            def chunk(k, carry2):
                pltpu.sync_copy(obuf, acc_g.at[iv.at[k]], add=True)
                return carry2

            lax.fori_loop(0, IB, chunk, 0)
            return carry
        return block

    @pl.when(c == 0)
    def _():
        for h in range(NC):
            lax.fori_loop(0, HB, mk_block(row_idx, h), 0)

    @pl.when(c == 1)
    def _():
        for h in range(NC):
            lax.fori_loop(0, HB, mk_block(col_idx, h), 0)

    pltpu.sync_copy(acc.at[s],
                    part_out.at[c, s // 2, :, pl.ds(G * (s % 2), G)])


_hist = functools.partial(
    pl.kernel,
    out_type=jax.ShapeDtypeStruct((NC, NS // 2, NPAD, 16), jnp.float32),
    mesh=_MESH,
    compiler_params=_SC_PARAMS,
    scratch_types=[
        pltpu.VMEM((IB, CW), jnp.int32),
        pltpu.VMEM((CW, G), jnp.float32),
        pltpu.VMEM_SHARED((NS, NPAD, G), jnp.float32),
    ],
)(_hist_body)


_R16 = GW // 16
_HR = _R16 // NS


def _hist_reduce_body(part_f, rr_out, rc_out, abuf, rbuf, sbuf, swsp):
    c = lax.axis_index("c")
    s = lax.axis_index("s")
    off = s * _HR
    pltpu.sync_copy(part_f.at[c, 0, pl.ds(off * 2, _HR * 2)], abuf)

    def add_tile(t, carry):
        pltpu.sync_copy(part_f.at[c, t, pl.ds(off * 2, _HR * 2)], rbuf)

        def vec(i, carry2):
            a0 = abuf[2 * i, :] + rbuf[2 * i, :]
            a1 = abuf[2 * i + 1, :] + rbuf[2 * i + 1, :]
            abuf[2 * i, :] = a0
            abuf[2 * i + 1, :] = a1
            return carry2

        lax.fori_loop(0, _HR, vec, 0)
        return carry

    lax.fori_loop(1, NS // 2, add_tile, 0)

    pltpu.sync_copy(abuf.at[:, pl.ds(0, G)], swsp.at[s, :, pl.ds(G, G)])
    pltpu.sync_copy(abuf.at[:, pl.ds(G, G)], swsp.at[s, :, pl.ds(0, G)])
    pltpu.sync_copy(swsp.at[s], sbuf)

    def recip(i, carry):
        v0 = abuf[2 * i, :] + sbuf[2 * i, :]
        v1 = abuf[2 * i + 1, :] + sbuf[2 * i + 1, :]
        abuf[2 * i, :] = jnp.where(v0 > 0, 1.0 / v0, 0.0)
        abuf[2 * i + 1, :] = jnp.where(v1 > 0, 1.0 / v1, 0.0)
        return carry

    lax.fori_loop(0, _HR, recip, 0)

    @pl.when(c == 0)
    def _():
        pltpu.sync_copy(abuf, rr_out.at[pl.ds(off * 2, _HR * 2)])

    @pl.when(c == 1)
    def _():
        pltpu.sync_copy(abuf, rc_out.at[pl.ds(off * 2, _HR * 2)])


_hist_reduce = functools.partial(
    pl.kernel,
    out_type=(
        jax.ShapeDtypeStruct((NPAD, 16), jnp.float32),
        jax.ShapeDtypeStruct((NPAD, 16), jnp.float32),
    ),
    mesh=_MESH,
    compiler_params=_SC_PARAMS,
    scratch_types=[
        pltpu.VMEM((_HR * 2, 16), jnp.float32),
        pltpu.VMEM((_HR * 2, 16), jnp.float32),
        pltpu.VMEM((_HR * 2, 16), jnp.float32),
        pltpu.VMEM_SHARED((NS, _HR * 2, 16), jnp.float32),
    ],
)(_hist_reduce_body)


_NBUF = IB


def _scatter_pass_body(tab, src_idx, dst_idx, zeros_g,
                       p_out,
                       iva_s, iva_d, ivb_s, ivb_d, gbuf, acc, *sems):
    c = lax.axis_index("c")
    s = lax.axis_index("s")
    pltpu.sync_copy(zeros_g, acc.at[s])
    tab_g = tab.at[s]
    acc_g = acc.at[s]
    gsems = sems[:_NBUF]
    isems = sems[_NBUF:]

    pltpu.sync_copy(src_idx.at[c, pl.ds(0, IB)], iva_s)
    pltpu.sync_copy(dst_idx.at[c, pl.ds(0, IB)], iva_d)

    def dblock(bb, carry):
        for par in range(2):
            b = 2 * bb + par
            cur_s, cur_d = (iva_s, iva_d) if par == 0 else (ivb_s, ivb_d)
            nxt_s, nxt_d = (ivb_s, ivb_d) if par == 0 else (iva_s, iva_d)

            @pl.when(b + 1 < PNB)
            def _():
                pltpu.async_copy(
                    src_idx.at[c, pl.ds((b + 1) * IB, IB)], nxt_s, isems[0])
                pltpu.async_copy(
                    dst_idx.at[c, pl.ds((b + 1) * IB, IB)], nxt_d, isems[1])

            descs = [
                pltpu.async_copy(tab_g.at[cur_s.at[j]], gbuf.at[j], gsems[j])
                for j in range(IB)
            ]
            for k in range(IB):
                descs[k].wait()
                pltpu.sync_copy(gbuf.at[k], acc_g.at[cur_d.at[k]], add=True)

            @pl.when(b + 1 < PNB)
            def _():
                pltpu.make_async_copy(
                    src_idx.at[c, pl.ds((b + 1) * IB, IB)], nxt_s,
                    isems[0]).wait()
                pltpu.make_async_copy(
                    dst_idx.at[c, pl.ds((b + 1) * IB, IB)], nxt_d,
                    isems[1]).wait()
        return carry

    lax.fori_loop(0, PNB // 2, dblock, 0)
    pltpu.sync_copy(acc.at[s],
                    p_out.at[c, s // 2, :, pl.ds(G * (s % 2), G)])


_scatter_pass = functools.partial(
    pl.kernel,
    out_type=jax.ShapeDtypeStruct((NC, NS // 2, NPAD, 16), jnp.float32),
    mesh=_MESH,
    compiler_params=_SC_PARAMS,
    scratch_types=[
        pltpu.VMEM((IB, CW), jnp.int32),
        pltpu.VMEM((IB, CW), jnp.int32),
        pltpu.VMEM((IB, CW), jnp.int32),
        pltpu.VMEM((IB, CW), jnp.int32),
        pltpu.VMEM((_NBUF, CW, G), jnp.float32),
        pltpu.VMEM_SHARED((NS, NPAD, G), jnp.float32),
    ] + [pltpu.SemaphoreType.DMA] * (_NBUF + 2),
)(_scatter_pass_body)


_MCH = 2
_MRW = NPAD // 4 // _MCH


def _pair_body(src4, recip16, extra, dst3, b0, b1, br, bo, bvec, with_bias):
    c = lax.axis_index("c")
    s = lax.axis_index("s")
    sp = s % (NS // 2)
    base = (c * 2 + s // (NS // 2)) * (NPAD // 4)
    if with_bias:
        pltpu.sync_copy(extra.at[sp], bvec)

    def sub(m, carry):
        off = base + m * _MRW
        sl = pl.ds(off, _MRW)
        pltpu.sync_copy(src4.at[0, sp, sl], b0)
        pltpu.sync_copy(src4.at[1, sp, sl], b1)
        pltpu.sync_copy(recip16.at[sl], br)
        if with_bias:
            bv = bvec[0, :]

        def vec(i, carry2):
            v = (b0[i, :] + b1[i, :]) * br[i, :]
            if with_bias:
                v = jnp.maximum(v + bv, 0.0)
            bo[i, :] = v
            return carry2

        lax.fori_loop(0, _MRW, vec, 0)
        pltpu.sync_copy(bo.at[:, pl.ds(0, G)], dst3.at[2 * sp, sl])
        pltpu.sync_copy(bo.at[:, pl.ds(G, G)], dst3.at[2 * sp + 1, sl])
        return carry

    lax.fori_loop(0, _MCH, sub, 0)


def _mk_pair(with_bias):
    body = functools.partial(_pair_body, with_bias=with_bias)
    scratch = [
        pltpu.VMEM((_MRW, 16), jnp.float32),
        pltpu.VMEM((_MRW, 16), jnp.float32),
        pltpu.VMEM((_MRW, 16), jnp.float32),
        pltpu.VMEM((_MRW, 16), jnp.float32),
        pltpu.VMEM((1, 16), jnp.float32),
    ]
    return functools.partial(
        pl.kernel,
        out_type=jax.ShapeDtypeStruct((NS, NPAD, G), jnp.float32),
        mesh=_MESH,
        compiler_params=_SC_PARAMS,
        scratch_types=scratch,
    )(body)


_mid = _mk_pair(False)
_finishf = _mk_pair(True)


def _assemble_body(osp3, out2, v2d):
    c = lax.axis_index("c")
    s = lax.axis_index("s")
    r0 = c * AR
    pltpu.sync_copy(osp3.at[s, pl.ds(r0, AR)], v2d)
    pltpu.sync_copy(v2d, out2.at[pl.ds(r0, AR), pl.ds(G * s, G)])


_assemble = functools.partial(
    pl.kernel,
    out_type=jax.ShapeDtypeStruct((N, F), jnp.float32),
    mesh=_MESH,
    compiler_params=_SC_PARAMS,
    scratch_types=[
        pltpu.VMEM((AR, G), jnp.float32),
    ],
)(_assemble_body)


def kernel(x, adj, W, bias):
    row_p = adj[0].astype(jnp.int32).reshape(NC, PCH, CW)
    col_p = adj[1].astype(jnp.int32).reshape(NC, PCH, CW)
    bias16 = bias.reshape(NS // 2, 1, 16)

    x_p = jnp.pad(x, ((0, NPAD - N), (0, 0)))
    zeros_g = jnp.zeros((NPAD, G), jnp.float32)
    ones_g = jnp.ones((CW, G), jnp.float32)

    hp = _hist(row_p, col_p, zeros_g, ones_g)
    recip_row, recip_col = _hist_reduce(hp)
    xt = _matmul(x_p, W)
    tab1 = _split(xt)
    p = _scatter_pass(tab1, row_p, col_p, zeros_g)
    e3 = _mid(p, recip_col, bias16)
    q = _scatter_pass(e3, col_p, row_p, zeros_g)
    osp = _finishf(q, recip_row, bias16)
    return _assemble(osp)


# speedup vs baseline: 1.0004x; 1.0004x over previous
import functools

import jax
import jax.numpy as jnp
from jax import lax
from jax.experimental import pallas as pl
from jax.experimental.pallas import tpu as pltpu
from jax.experimental.pallas import tpu_sc as plsc

N = 10000
NPAD = 10240
F = 128
NNZ = 320000
NC = 2
NS = 16
G = 8
CW = 125
IB = 16
PCH = NNZ // NC // CW
PNB = PCH // IB
HB = PCH // NS // IB
GW = NPAD * G
HROW = NPAD // NC
AR = N // NC

_MESH = plsc.VectorSubcoreMesh(core_axis_name="c", subcore_axis_name="s",
                               num_cores=NC, num_subcores=NS)
_SC_PARAMS = pltpu.CompilerParams(use_tc_tiling_on_sc=False)


def _matmul_body(x_ref, w_ref, o_ref):
    o_ref[...] = lax.dot_general(
        x_ref[...], w_ref[...], (((1,), (1,)), ((), ())),
        preferred_element_type=jnp.float32)


def _matmul(x, w):
    return pl.pallas_call(
        _matmul_body,
        out_shape=jax.ShapeDtypeStruct((NPAD, F), jnp.float32),
    )(x, w)


def _split_body(xt2, tab3, v2d):
    c = lax.axis_index("c")
    s = lax.axis_index("s")
    r0 = c * HROW
    pltpu.sync_copy(xt2.at[pl.ds(r0, HROW), pl.ds(G * s, G)], v2d)
    pltpu.sync_copy(v2d, tab3.at[s, pl.ds(r0, HROW)])


_split = functools.partial(
    pl.kernel,
    out_type=jax.ShapeDtypeStruct((NS, NPAD, G), jnp.float32),
    mesh=_MESH,
    compiler_params=_SC_PARAMS,
    scratch_types=[
        pltpu.VMEM((HROW, G), jnp.float32),
    ],
)(_split_body)


def _hist_body(row_idx, col_idx, zeros_g, ones_g, part_out, iv, obuf, acc):
    c = lax.axis_index("c")
    s = lax.axis_index("s")
    pltpu.sync_copy(zeros_g, acc.at[s])
    pltpu.sync_copy(ones_g, obuf)
    acc_g = acc.at[s]

    def mk_block(idx, h):
        def block(b, carry):
            pltpu.sync_copy(idx.at[h, pl.ds(s * (PCH // NS) + b * IB, IB)],
                            iv)

            def chunk(k, carry2):
                pltpu.sync_copy(obuf, acc_g.at[iv.at[k]], add=True)
                return carry2

            lax.fori_loop(0, IB, chunk, 0)
            return carry
        return block

    @pl.when(c == 0)
    def _():
        for h in range(NC):
            lax.fori_loop(0, HB, mk_block(row_idx, h), 0)

    @pl.when(c == 1)
    def _():
        for h in range(NC):
            lax.fori_loop(0, HB, mk_block(col_idx, h), 0)

    pltpu.sync_copy(acc.at[s],
                    part_out.at[c, s // 2, :, pl.ds(G * (s % 2), G)])


_hist = functools.partial(
    pl.kernel,
    out_type=jax.ShapeDtypeStruct((NC, NS // 2, NPAD, 16), jnp.float32),
    mesh=_MESH,
    compiler_params=_SC_PARAMS,
    scratch_types=[
        pltpu.VMEM((IB, CW), jnp.int32),
        pltpu.VMEM((CW, G), jnp.float32),
        pltpu.VMEM_SHARED((NS, NPAD, G), jnp.float32),
    ],
)(_hist_body)


_R16 = GW // 16
_HR = _R16 // NS


def _hist_reduce_body(part_f, rr_out, rc_out, abuf, rbuf, sbuf, swsp):
    c = lax.axis_index("c")
    s = lax.axis_index("s")
    off = s * _HR
    pltpu.sync_copy(part_f.at[c, 0, pl.ds(off * 2, _HR * 2)], abuf)

    def add_tile(t, carry):
        pltpu.sync_copy(part_f.at[c, t, pl.ds(off * 2, _HR * 2)], rbuf)

        def vec(i, carry2):
            a0 = abuf[2 * i, :] + rbuf[2 * i, :]
            a1 = abuf[2 * i + 1, :] + rbuf[2 * i + 1, :]
            abuf[2 * i, :] = a0
            abuf[2 * i + 1, :] = a1
            return carry2

        lax.fori_loop(0, _HR, vec, 0)
        return carry

    lax.fori_loop(1, NS // 2, add_tile, 0)

    pltpu.sync_copy(abuf.at[:, pl.ds(0, G)], swsp.at[s, :, pl.ds(G, G)])
    pltpu.sync_copy(abuf.at[:, pl.ds(G, G)], swsp.at[s, :, pl.ds(0, G)])
    pltpu.sync_copy(swsp.at[s], sbuf)

    def recip(i, carry):
        v0 = abuf[2 * i, :] + sbuf[2 * i, :]
        v1 = abuf[2 * i + 1, :] + sbuf[2 * i + 1, :]
        abuf[2 * i, :] = jnp.where(v0 > 0, 1.0 / v0, 0.0)
        abuf[2 * i + 1, :] = jnp.where(v1 > 0, 1.0 / v1, 0.0)
        return carry

    lax.fori_loop(0, _HR, recip, 0)

    @pl.when(c == 0)
    def _():
        pltpu.sync_copy(abuf, rr_out.at[pl.ds(off * 2, _HR * 2)])

    @pl.when(c == 1)
    def _():
        pltpu.sync_copy(abuf, rc_out.at[pl.ds(off * 2, _HR * 2)])


_hist_reduce = functools.partial(
    pl.kernel,
    out_type=(
        jax.ShapeDtypeStruct((NPAD, 16), jnp.float32),
        jax.ShapeDtypeStruct((NPAD, 16), jnp.float32),
    ),
    mesh=_MESH,
    compiler_params=_SC_PARAMS,
    scratch_types=[
        pltpu.VMEM((_HR * 2, 16), jnp.float32),
        pltpu.VMEM((_HR * 2, 16), jnp.float32),
        pltpu.VMEM((_HR * 2, 16), jnp.float32),
        pltpu.VMEM_SHARED((NS, _HR * 2, 16), jnp.float32),
    ],
)(_hist_reduce_body)


_NBUF = IB


def _scatter_pass_body(tab, src_idx, dst_idx, zeros_g,
                       p_out,
                       iva_s, iva_d, ivb_s, ivb_d, gbuf, acc, *sems):
    c = lax.axis_index("c")
    s = lax.axis_index("s")
    pltpu.sync_copy(zeros_g, acc.at[s])
    tab_g = tab.at[s]
    acc_g = acc.at[s]
    gsems = sems[:_NBUF]
    isems = sems[_NBUF:]

    pltpu.sync_copy(src_idx.at[c, pl.ds(0, IB)], iva_s)
    pltpu.sync_copy(dst_idx.at[c, pl.ds(0, IB)], iva_d)

    def dblock(bb, carry):
        for par in range(2):
            b = 2 * bb + par
            cur_s, cur_d = (iva_s, iva_d) if par == 0 else (ivb_s, ivb_d)
            nxt_s, nxt_d = (ivb_s, ivb_d) if par == 0 else (iva_s, iva_d)

            @pl.when(b + 1 < PNB)
            def _():
                pltpu.async_copy(
                    src_idx.at[c, pl.ds((b + 1) * IB, IB)], nxt_s, isems[0])
                pltpu.async_copy(
                    dst_idx.at[c, pl.ds((b + 1) * IB, IB)], nxt_d, isems[1])

            descs = [
                pltpu.async_copy(tab_g.at[cur_s.at[j]], gbuf.at[j], gsems[j])
                for j in range(IB)
            ]
            for k in range(IB):
                descs[k].wait()
                pltpu.sync_copy(gbuf.at[k], acc_g.at[cur_d.at[k]], add=True)

            @pl.when(b + 1 < PNB)
            def _():
                pltpu.make_async_copy(
                    src_idx.at[c, pl.ds((b + 1) * IB, IB)], nxt_s,
                    isems[0]).wait()
                pltpu.make_async_copy(
                    dst_idx.at[c, pl.ds((b + 1) * IB, IB)], nxt_d,
                    isems[1]).wait()
        return carry

    lax.fori_loop(0, PNB // 2, dblock, 0)
    pltpu.sync_copy(acc.at[s],
                    p_out.at[c, s // 2, :, pl.ds(G * (s % 2), G)])


_scatter_pass = functools.partial(
    pl.kernel,
    out_type=jax.ShapeDtypeStruct((NC, NS // 2, NPAD, 16), jnp.float32),
    mesh=_MESH,
    compiler_params=_SC_PARAMS,
    scratch_types=[
        pltpu.VMEM((IB, CW), jnp.int32),
        pltpu.VMEM((IB, CW), jnp.int32),
        pltpu.VMEM((IB, CW), jnp.int32),
        pltpu.VMEM((IB, CW), jnp.int32),
        pltpu.VMEM((_NBUF, CW, G), jnp.float32),
        pltpu.VMEM_SHARED((NS, NPAD, G), jnp.float32),
    ] + [pltpu.SemaphoreType.DMA] * (_NBUF + 2),
)(_scatter_pass_body)


_MCH = 2
_MRW = NPAD // 4 // _MCH


def _pair_body(src4, recip16, extra, dst3, b0, b1, br, bo, bvec, with_bias):
    c = lax.axis_index("c")
    s = lax.axis_index("s")
    sp = s % (NS // 2)
    base = (c * 2 + s // (NS // 2)) * (NPAD // 4)
    if with_bias:
        pltpu.sync_copy(extra.at[sp], bvec)

    def sub(m, carry):
        off = base + m * _MRW
        sl = pl.ds(off, _MRW)
        pltpu.sync_copy(src4.at[0, sp, sl], b0)
        pltpu.sync_copy(src4.at[1, sp, sl], b1)
        pltpu.sync_copy(recip16.at[sl], br)
        if with_bias:
            bv = bvec[0, :]

        def vec(i, carry2):
            v = (b0[i, :] + b1[i, :]) * br[i, :]
            if with_bias:
                v = jnp.maximum(v + bv, 0.0)
            bo[i, :] = v
            return carry2

        lax.fori_loop(0, _MRW, vec, 0)
        pltpu.sync_copy(bo.at[:, pl.ds(0, G)], dst3.at[2 * sp, sl])
        pltpu.sync_copy(bo.at[:, pl.ds(G, G)], dst3.at[2 * sp + 1, sl])
        return carry

    lax.fori_loop(0, _MCH, sub, 0)


def _mk_pair(with_bias):
    body = functools.partial(_pair_body, with_bias=with_bias)
    scratch = [
        pltpu.VMEM((_MRW, 16), jnp.float32),
        pltpu.VMEM((_MRW, 16), jnp.float32),
        pltpu.VMEM((_MRW, 16), jnp.float32),
        pltpu.VMEM((_MRW, 16), jnp.float32),
        pltpu.VMEM((1, 16), jnp.float32),
    ]
    return functools.partial(
        pl.kernel,
        out_type=jax.ShapeDtypeStruct((NS, NPAD, G), jnp.float32),
        mesh=_MESH,
        compiler_params=_SC_PARAMS,
        scratch_types=scratch,
    )(body)


_mid = _mk_pair(False)
_finishf = _mk_pair(True)


def _assemble_body(osp3, out2, v2d):
    c = lax.axis_index("c")
    s = lax.axis_index("s")
    r0 = c * AR
    pltpu.sync_copy(osp3.at[s, pl.ds(r0, AR)], v2d)
    pltpu.sync_copy(v2d, out2.at[pl.ds(r0, AR), pl.ds(G * s, G)])


_assemble = functools.partial(
    pl.kernel,
    out_type=jax.ShapeDtypeStruct((N, F), jnp.float32),
    mesh=_MESH,
    compiler_params=_SC_PARAMS,
    scratch_types=[
        pltpu.VMEM((AR, G), jnp.float32),
    ],
)(_assemble_body)


def kernel(x, adj, W, bias):
    row_p = adj[0].astype(jnp.int32).reshape(NC, PCH, CW)
    col_p = adj[1].astype(jnp.int32).reshape(NC, PCH, CW)
    bias16 = bias.reshape(NS // 2, 1, 16)

    x_p = jnp.pad(x, ((0, NPAD - N), (0, 0)))
    zeros_g = jnp.zeros((NPAD, G), jnp.float32)
    ones_g = jnp.ones((CW, G), jnp.float32)

    hp = _hist(row_p, col_p, zeros_g, ones_g)
    recip_row, recip_col = _hist_reduce(hp)
    xt = _matmul(x_p, W)
    tab1 = _split(xt)
    p = _scatter_pass(tab1, row_p, col_p, zeros_g)
    e3 = _mid(p, recip_col, bias16)
    q = _scatter_pass(e3, col_p, row_p, zeros_g)
    osp = _finishf(q, recip_row, bias16)
    return _assemble(osp)
